# baseline (speedup 1.0000x reference)
import functools

import jax
import jax.numpy as jnp
from jax import lax
from jax.experimental import pallas as pl
from jax.experimental.pallas import tpu as pltpu
from jax.experimental.pallas import tpu_sc as plsc

_NC = 2
_NS = 16
_NW = _NC * _NS
_L = 16

_N = 4 * 8192
_D = 32
_H = 16
_T = _N // _NW


def _sc_body(emb_hbm, w1_hbm, b1_hbm, w2_hbm, b2_hbm, out_hbm,
             emb_v, w1_v, b1_v, w2_v, b2_v, out_v):
    wid = lax.axis_index("s") * _NC + lax.axis_index("c")
    base = wid * _T
    pltpu.sync_copy(emb_hbm.at[pl.ds(base * _D, _T * _D)], emb_v)
    pltpu.sync_copy(w1_hbm, w1_v)
    pltpu.sync_copy(b1_hbm, b1_v)
    pltpu.sync_copy(w2_hbm, w2_v)
    pltpu.sync_copy(b2_hbm, b2_v)
    w1r = [w1_v[pl.ds(16 * i, 16)] for i in range(_H * _D // 16)]
    b1r = b1_v[...]
    w2r = w2_v[...]

    b2vec = b2_v[...]
    lane = lax.iota(jnp.int32, _L) * _D

    def body(g, carry):
        t0 = g * _L
        idx = lane + t0 * _D
        xs = [plsc.load_gather(emb_v, [idx + d]) for d in range(_D)]
        z = b2vec
        for j in range(_H):
            acc = jnp.broadcast_to(b1r[j], (_L,))
            for d in range(_D):
                acc = acc + xs[d] * w1r[(j * _D + d) // 16][d % 16]
            h = jnp.maximum(acc, 0.0)
            z = z + h * w2r[j]
        s = 1.0 / (1.0 + jnp.exp(-z))
        out_v[pl.ds(t0, _L)] = s
        return carry

    lax.fori_loop(0, _T // _L, body, 0)
    pltpu.sync_copy(out_v, out_hbm.at[pl.ds(base, _T)])


@jax.jit
def _run(flat_emb, w1, b1, w2, b2v):
    mesh = plsc.VectorSubcoreMesh(core_axis_name="c", subcore_axis_name="s")
    return pl.kernel(
        _sc_body,
        out_type=jax.ShapeDtypeStruct((_N,), jnp.float32),
        mesh=mesh,
        compiler_params=pltpu.CompilerParams(needs_layout_passes=False),
        scratch_types=[
            pltpu.VMEM((_T * _D,), jnp.float32),
            pltpu.VMEM((_H * _D,), jnp.float32),
            pltpu.VMEM((_H,), jnp.float32),
            pltpu.VMEM((_H,), jnp.float32),
            pltpu.VMEM((_L,), jnp.float32),
            pltpu.VMEM((_T,), jnp.float32),
        ],
    )(flat_emb, w1, b1, w2, b2v)


def kernel(embeddings, W1, b1, W2, b2):
    bsz, seq, _ = embeddings.shape
    flat = embeddings.reshape(-1)
    w1 = W1.reshape(-1)
    w2 = W2.reshape(-1)
    b2v = jnp.broadcast_to(b2, (_L,)).astype(jnp.float32)
    out = _run(flat, w1, b1, w2, b2v)
    return out.reshape(bsz, seq)


# speedup vs baseline: 1.3444x; 1.3444x over previous
import jax
import jax.numpy as jnp
from jax import lax
from jax.experimental import pallas as pl
from jax.experimental.pallas import tpu as pltpu
from jax.experimental.pallas import tpu_sc as plsc

_NC = 2
_NS = 16
_NW = _NC * _NS
_L = 16

_N = 4 * 8192
_D = 32
_H = 16
_T = _N // _NW
_G = 2


def _sc_body(emb_hbm, ws_hbm, b1s_hbm, w2s_hbm, b2s_hbm, out_hbm,
             emb_v, ws_v, b1s_v, w2s_v, b2s_v, out_v):
    wid = lax.axis_index("s") * _NC + lax.axis_index("c")
    base = wid * _T
    pltpu.sync_copy(emb_hbm.at[pl.ds(base * _D, _T * _D)], emb_v)
    pltpu.sync_copy(ws_hbm, ws_v)
    pltpu.sync_copy(b1s_hbm, b1s_v)
    pltpu.sync_copy(w2s_hbm, w2s_v)
    pltpu.sync_copy(b2s_hbm, b2s_v)

    b2vec = b2s_v[...]
    lane = lax.iota(jnp.int32, _L) * _D

    def body(i, carry):
        t0 = i * (_L * _G)
        idx = [lane + (t0 + g * _L) * _D for g in range(_G)]
        hs = [[b1s_v[pl.ds(_L * j, _L)] for g in range(_G)] for j in range(_H)]
        for d in range(_D):
            xs = [plsc.load_gather(emb_v, [idx[g] + d]) for g in range(_G)]
            for j in range(_H):
                w = ws_v[pl.ds((j * _D + d) * _L, _L)]
                for g in range(_G):
                    hs[j][g] = hs[j][g] + xs[g] * w
        zs = [b2vec for g in range(_G)]
        for j in range(_H):
            w2 = w2s_v[pl.ds(_L * j, _L)]
            for g in range(_G):
                zs[g] = zs[g] + jnp.maximum(hs[j][g], 0.0) * w2
        for g in range(_G):
            s = 1.0 / (1.0 + jnp.exp(-zs[g]))
            out_v[pl.ds(t0 + g * _L, _L)] = s
        return carry

    lax.fori_loop(0, _T // (_L * _G), body, 0)
    pltpu.sync_copy(out_v, out_hbm.at[pl.ds(base, _T)])


@jax.jit
def _run(flat_emb, ws, b1s, w2s, b2s):
    mesh = plsc.VectorSubcoreMesh(core_axis_name="c", subcore_axis_name="s")
    return pl.kernel(
        _sc_body,
        out_type=jax.ShapeDtypeStruct((_N,), jnp.float32),
        mesh=mesh,
        compiler_params=pltpu.CompilerParams(needs_layout_passes=False),
        scratch_types=[
            pltpu.VMEM((_T * _D,), jnp.float32),
            pltpu.VMEM((_H * _D * _L,), jnp.float32),
            pltpu.VMEM((_H * _L,), jnp.float32),
            pltpu.VMEM((_H * _L,), jnp.float32),
            pltpu.VMEM((_L,), jnp.float32),
            pltpu.VMEM((_T,), jnp.float32),
        ],
    )(flat_emb, ws, b1s, w2s, b2s)


def kernel(embeddings, W1, b1, W2, b2):
    bsz, seq, _ = embeddings.shape
    flat = embeddings.reshape(-1)
    ws = jnp.repeat(W1.reshape(-1), _L)
    b1s = jnp.repeat(b1, _L)
    w2s = jnp.repeat(W2.reshape(-1), _L)
    b2s = jnp.broadcast_to(b2, (_L,)).astype(jnp.float32)
    out = _run(flat, ws, b1s, w2s, b2s)
    return out.reshape(bsz, seq)
